# baseline (speedup 1.0000x reference)
import functools

import jax
import jax.numpy as jnp
from jax.experimental import pallas as pl
from jax.experimental.pallas import tpu as pltpu

_MIN_SIGMA = 1.0
_BN_EPS = 1e-5


def _pass1_body(xr_ref, st_ref, mu_ref, ls_ref, kemb_ref, semb_ref, pt_ref,
                pb_ref, wcat_ref, bcat_ref, w1t_ref, b1_ref, w2t_ref, b2_ref,
                h_ref, stats_ref, film_ref, seq_scr, im2_scr, *,
                rows, seq_len, emb_dim, hidden):
    f32 = jnp.float32
    bf16 = jnp.bfloat16
    sig = jax.nn.softplus(ls_ref[...]) + _MIN_SIGMA
    neg_half_inv = -0.5 / (sig * sig)
    for r in range(rows):
        x = xr_ref[r]
        xa = jnp.abs(x)
        d = xa - mu_ref[...]
        w = jnp.exp(d * d * neg_half_inv)
        s = jnp.sum(w, axis=-1, keepdims=True)
        wn = w / (s + 1e-8)
        mag = jnp.dot(wn.astype(bf16), kemb_ref[...],
                      preferred_element_type=f32)
        s_emb = jnp.where(x < 0.0, semb_ref[1:2, :],
                          jnp.where(x > 0.0, semb_ref[2:3, :],
                                    semb_ref[0:1, :]))
        emb = mag + s_emb
        seq = jnp.dot(emb.astype(bf16), pt_ref[...],
                      preferred_element_type=f32) + pb_ref[...]
        seq_scr[r, 0:4, :] = jnp.zeros((4, emb_dim), f32)
        seq_scr[r, 4:4 + seq_len, :] = seq
        seq_scr[r, 4 + seq_len:8 + seq_len, :] = jnp.zeros((4, emb_dim), f32)
        for o in range(7):
            im2_scr[r, :, o * emb_dim:(o + 1) * emb_dim] = (
                seq_scr[r, 1 + o:1 + o + seq_len, :].astype(bf16))
        h = jnp.dot(im2_scr[r], wcat_ref[...], preferred_element_type=f32)
        h = jnp.maximum(h + bcat_ref[...], 0.0)
        h_ref[r] = h
        s1 = jnp.sum(h, axis=0, keepdims=True)
        s2 = jnp.sum(h * h, axis=0, keepdims=True)
        stats_ref[r] = jnp.concatenate(
            [s1, s2, jnp.zeros((6, hidden), f32)], axis=0)
        f1 = jnp.maximum(jnp.dot(st_ref[r].astype(bf16), w1t_ref[...],
                                 preferred_element_type=f32)
                         + b1_ref[...], 0.0)
        film_ref[r] = jnp.dot(f1.astype(bf16), w2t_ref[...],
                              preferred_element_type=f32) + b2_ref[...]


def _pass2_body(h_ref, stats_ref, film_ref, bng_ref, bnb_ref, aw_ref,
                o_ref, *, rows, n_total, hidden):
    f32 = jnp.float32
    bf16 = jnp.bfloat16
    ps = jnp.sum(stats_ref[...], axis=0)
    s1 = ps[0:1, :]
    s2 = ps[1:2, :]
    inv_n = 1.0 / n_total
    mean = s1 * inv_n
    var = s2 * inv_n - mean * mean
    a = bng_ref[...] * jax.lax.rsqrt(var + _BN_EPS)
    bsh = bnb_ref[...] - mean * a
    for r in range(rows):
        f = film_ref[r]
        gamma = f[:, :hidden]
        beta = f[:, hidden:]
        a2 = a * (1.0 + gamma)
        b2 = bsh * (1.0 + gamma) + beta
        hm = h_ref[r] * a2 + b2
        hmb = hm.astype(bf16)
        scr = jax.lax.dot_general(aw_ref[...], hmb, (((1,), (1,)), ((), ())),
                                  preferred_element_type=f32)
        m = jnp.max(scr, axis=1, keepdims=True)
        e = jnp.exp(scr - m)
        den = jnp.sum(e, axis=1, keepdims=True)
        wgt = e / den
        wgt_c = jnp.transpose(wgt)
        pooled = jnp.sum(hm * wgt_c, axis=0, keepdims=True)
        o_ref[r] = pooled


@functools.partial(jax.jit, static_argnames=("interpret",))
def kernel(raw_seq, raw_stats, mu, log_sigma, kernel_emb, sign_emb, proj_w,
           proj_b, conv3_w, conv3_b, conv5_w, conv5_b, conv7_w, conv7_b,
           bn_gamma, bn_beta, stat_w1, stat_b1, stat_w2, stat_b2, attn_w,
           attn_b, interpret=False):
    f32 = jnp.float32
    bf16 = jnp.bfloat16
    bsz, seq_len = raw_seq.shape
    n_kern, emb_dim = kernel_emb.shape
    c3 = conv3_w.shape[0]
    c5 = conv5_w.shape[0]
    c7 = conv7_w.shape[0]
    hidden = c3 + c5 + c7
    stat_dim = raw_stats.shape[1]
    mlp_hid = stat_w1.shape[0]

    xr = raw_seq.reshape(bsz, seq_len, 1)
    st = raw_stats.reshape(bsz, 1, stat_dim)
    mu2 = mu.reshape(1, n_kern)
    ls2 = log_sigma.reshape(1, n_kern)
    pt = proj_w.T.astype(bf16)
    pb = proj_b.reshape(1, emb_dim)
    blocks = []
    for o in range(-3, 4):
        parts = []
        for w_c, k in ((conv3_w, 3), (conv5_w, 5), (conv7_w, 7)):
            p = k // 2
            t = o + p
            if 0 <= t < k:
                parts.append(w_c[:, :, t].T)
            else:
                parts.append(jnp.zeros((emb_dim, w_c.shape[0]), f32))
        blocks.append(jnp.concatenate(parts, axis=1))
    wcat = jnp.concatenate(blocks, axis=0).astype(bf16)
    bcat = jnp.concatenate([conv3_b, conv5_b, conv7_b]).reshape(1, hidden)
    w1t = stat_w1.T.astype(bf16)
    b1 = stat_b1.reshape(1, mlp_hid)
    w2t = stat_w2.T.astype(bf16)
    b2 = stat_b2.reshape(1, 2 * hidden)
    bng = bn_gamma.reshape(1, hidden)
    bnb = bn_beta.reshape(1, hidden)
    aw = attn_w.reshape(1, hidden).astype(bf16)

    r1 = 2
    r2 = 4
    whole = lambda shape: pl.BlockSpec(shape, lambda b: tuple(0 for _ in shape))
    row3 = lambda g, s1, s2: pl.BlockSpec((g, s1, s2), lambda b: (b, 0, 0))

    h, stats, film = pl.pallas_call(
        functools.partial(_pass1_body, rows=r1, seq_len=seq_len,
                          emb_dim=emb_dim, hidden=hidden),
        grid=(bsz // r1,),
        in_specs=[
            row3(r1, seq_len, 1),
            row3(r1, 1, stat_dim),
            whole((1, n_kern)), whole((1, n_kern)),
            whole((n_kern, emb_dim)), whole((3, emb_dim)),
            whole((emb_dim, emb_dim)), whole((1, emb_dim)),
            whole((7 * emb_dim, hidden)), whole((1, hidden)),
            whole((stat_dim, mlp_hid)), whole((1, mlp_hid)),
            whole((mlp_hid, 2 * hidden)), whole((1, 2 * hidden)),
        ],
        out_specs=[row3(r1, seq_len, hidden), row3(r1, 8, hidden),
                   row3(r1, 1, 2 * hidden)],
        out_shape=[
            jax.ShapeDtypeStruct((bsz, seq_len, hidden), f32),
            jax.ShapeDtypeStruct((bsz, 8, hidden), f32),
            jax.ShapeDtypeStruct((bsz, 1, 2 * hidden), f32),
        ],
        scratch_shapes=[
            pltpu.VMEM((r1, seq_len + 8, emb_dim), f32),
            pltpu.VMEM((r1, seq_len, 7 * emb_dim), bf16),
        ],
        compiler_params=pltpu.CompilerParams(
            dimension_semantics=("arbitrary",),
        ),
        name="traffic_node_pass1",
        interpret=interpret,
    )(xr, st, mu2, ls2, kernel_emb.astype(bf16), sign_emb, pt, pb, wcat,
      bcat, w1t, b1, w2t, b2)

    out = pl.pallas_call(
        functools.partial(_pass2_body, rows=r2, n_total=float(bsz * seq_len),
                          hidden=hidden),
        grid=(bsz // r2,),
        in_specs=[
            row3(r2, seq_len, hidden),
            pl.BlockSpec((bsz, 8, hidden), lambda b: (0, 0, 0)),
            row3(r2, 1, 2 * hidden),
            whole((1, hidden)), whole((1, hidden)), whole((1, hidden)),
        ],
        out_specs=row3(r2, 1, hidden),
        out_shape=jax.ShapeDtypeStruct((bsz, 1, hidden), f32),
        compiler_params=pltpu.CompilerParams(
            dimension_semantics=("arbitrary",),
        ),
        name="traffic_node_pass2",
        interpret=interpret,
    )(h, stats, film, bng, bnb, aw)

    return out.reshape(bsz, hidden)


# speedup vs baseline: 1.1156x; 1.1156x over previous
import functools

import jax
import jax.numpy as jnp
from jax.experimental import pallas as pl
from jax.experimental.pallas import tpu as pltpu

_MIN_SIGMA = 1.0
_BN_EPS = 1e-5


def _pass1_body(xr_ref, st_ref, mu_ref, ls_ref, kemb_ref, semb_ref, pt_ref,
                pb_ref, wcat_ref, bcat_ref, w1t_ref, b1_ref, w2t_ref, b2_ref,
                h_ref, stats_ref, film_ref, seq_scr, im2_scr, *,
                rows, seq_len, emb_dim, hidden):
    f32 = jnp.float32
    bf16 = jnp.bfloat16
    sig = jax.nn.softplus(ls_ref[...]) + _MIN_SIGMA
    neg_half_inv = -0.5 / (sig * sig)
    for r in range(rows):
        x = xr_ref[r]
        xa = jnp.abs(x)
        d = xa - mu_ref[...]
        w = jnp.exp(d * d * neg_half_inv)
        s = jnp.sum(w, axis=-1, keepdims=True)
        wn = w / (s + 1e-8)
        mag = jnp.dot(wn.astype(bf16), kemb_ref[...],
                      preferred_element_type=f32)
        s_emb = jnp.where(x < 0.0, semb_ref[1:2, :],
                          jnp.where(x > 0.0, semb_ref[2:3, :],
                                    semb_ref[0:1, :]))
        emb = mag + s_emb
        seq = jnp.dot(emb.astype(bf16), pt_ref[...],
                      preferred_element_type=f32) + pb_ref[...]
        seq_scr[r, 0:4, :] = jnp.zeros((4, emb_dim), f32)
        seq_scr[r, 4:4 + seq_len, :] = seq
        seq_scr[r, 4 + seq_len:8 + seq_len, :] = jnp.zeros((4, emb_dim), f32)
        for o in range(7):
            im2_scr[r, :, o * emb_dim:(o + 1) * emb_dim] = (
                seq_scr[r, 1 + o:1 + o + seq_len, :].astype(bf16))
        h = jnp.dot(im2_scr[r], wcat_ref[...], preferred_element_type=f32)
        h = jnp.maximum(h + bcat_ref[...], 0.0)
        h_ref[r] = h
        s1 = jnp.sum(h, axis=0, keepdims=True)
        s2 = jnp.sum(h * h, axis=0, keepdims=True)
        stats_ref[r] = jnp.concatenate(
            [s1, s2, jnp.zeros((6, hidden), f32)], axis=0)
        f1 = jnp.maximum(jnp.dot(st_ref[r].astype(bf16), w1t_ref[...],
                                 preferred_element_type=f32)
                         + b1_ref[...], 0.0)
        film_ref[r] = jnp.dot(f1.astype(bf16), w2t_ref[...],
                              preferred_element_type=f32) + b2_ref[...]


def _pass2_body(h_ref, stats_ref, film_ref, bng_ref, bnb_ref, aw_ref,
                o_ref, *, rows, n_total, hidden):
    f32 = jnp.float32
    bf16 = jnp.bfloat16
    ps = jnp.sum(stats_ref[...], axis=0)
    s1 = ps[0:1, :]
    s2 = ps[1:2, :]
    inv_n = 1.0 / n_total
    mean = s1 * inv_n
    var = s2 * inv_n - mean * mean
    a = bng_ref[...] * jax.lax.rsqrt(var + _BN_EPS)
    bsh = bnb_ref[...] - mean * a
    for r in range(rows):
        f = film_ref[r]
        gamma = f[:, :hidden]
        beta = f[:, hidden:]
        a2 = a * (1.0 + gamma)
        b2 = bsh * (1.0 + gamma) + beta
        hm = h_ref[r] * a2 + b2
        hmb = hm.astype(bf16)
        scr = jax.lax.dot_general(aw_ref[...], hmb, (((1,), (1,)), ((), ())),
                                  preferred_element_type=f32)
        m = jnp.max(scr, axis=1, keepdims=True)
        e = jnp.exp(scr - m)
        den = jnp.sum(e, axis=1, keepdims=True)
        wgt = e / den
        wgt_c = jnp.transpose(wgt)
        pooled = jnp.sum(hm * wgt_c, axis=0, keepdims=True)
        o_ref[r] = pooled


@functools.partial(jax.jit, static_argnames=("interpret",))
def kernel(raw_seq, raw_stats, mu, log_sigma, kernel_emb, sign_emb, proj_w,
           proj_b, conv3_w, conv3_b, conv5_w, conv5_b, conv7_w, conv7_b,
           bn_gamma, bn_beta, stat_w1, stat_b1, stat_w2, stat_b2, attn_w,
           attn_b, interpret=False):
    f32 = jnp.float32
    bf16 = jnp.bfloat16
    bsz, seq_len = raw_seq.shape
    n_kern, emb_dim = kernel_emb.shape
    c3 = conv3_w.shape[0]
    c5 = conv5_w.shape[0]
    c7 = conv7_w.shape[0]
    hidden = c3 + c5 + c7
    stat_dim = raw_stats.shape[1]
    mlp_hid = stat_w1.shape[0]

    xr = raw_seq.reshape(bsz, seq_len, 1)
    st = raw_stats.reshape(bsz, 1, stat_dim)
    mu2 = mu.reshape(1, n_kern)
    ls2 = log_sigma.reshape(1, n_kern)
    pt = proj_w.T.astype(bf16)
    pb = proj_b.reshape(1, emb_dim)
    blocks = []
    for o in range(-3, 4):
        parts = []
        for w_c, k in ((conv3_w, 3), (conv5_w, 5), (conv7_w, 7)):
            p = k // 2
            t = o + p
            if 0 <= t < k:
                parts.append(w_c[:, :, t].T)
            else:
                parts.append(jnp.zeros((emb_dim, w_c.shape[0]), f32))
        blocks.append(jnp.concatenate(parts, axis=1))
    wcat = jnp.concatenate(blocks, axis=0).astype(bf16)
    bcat = jnp.concatenate([conv3_b, conv5_b, conv7_b]).reshape(1, hidden)
    w1t = stat_w1.T.astype(bf16)
    b1 = stat_b1.reshape(1, mlp_hid)
    w2t = stat_w2.T.astype(bf16)
    b2 = stat_b2.reshape(1, 2 * hidden)
    bng = bn_gamma.reshape(1, hidden)
    bnb = bn_beta.reshape(1, hidden)
    aw = attn_w.reshape(1, hidden).astype(bf16)

    r1 = 4
    r2 = 8
    whole = lambda shape: pl.BlockSpec(shape, lambda b: tuple(0 for _ in shape))
    row3 = lambda g, s1, s2: pl.BlockSpec((g, s1, s2), lambda b: (b, 0, 0))

    h, stats, film = pl.pallas_call(
        functools.partial(_pass1_body, rows=r1, seq_len=seq_len,
                          emb_dim=emb_dim, hidden=hidden),
        grid=(bsz // r1,),
        in_specs=[
            row3(r1, seq_len, 1),
            row3(r1, 1, stat_dim),
            whole((1, n_kern)), whole((1, n_kern)),
            whole((n_kern, emb_dim)), whole((3, emb_dim)),
            whole((emb_dim, emb_dim)), whole((1, emb_dim)),
            whole((7 * emb_dim, hidden)), whole((1, hidden)),
            whole((stat_dim, mlp_hid)), whole((1, mlp_hid)),
            whole((mlp_hid, 2 * hidden)), whole((1, 2 * hidden)),
        ],
        out_specs=[row3(r1, seq_len, hidden), row3(r1, 8, hidden),
                   row3(r1, 1, 2 * hidden)],
        out_shape=[
            jax.ShapeDtypeStruct((bsz, seq_len, hidden), f32),
            jax.ShapeDtypeStruct((bsz, 8, hidden), f32),
            jax.ShapeDtypeStruct((bsz, 1, 2 * hidden), f32),
        ],
        scratch_shapes=[
            pltpu.VMEM((r1, seq_len + 8, emb_dim), f32),
            pltpu.VMEM((r1, seq_len, 7 * emb_dim), bf16),
        ],
        compiler_params=pltpu.CompilerParams(
            dimension_semantics=("arbitrary",),
            vmem_limit_bytes=56 * 1024 * 1024,
        ),
        name="traffic_node_pass1",
        interpret=interpret,
    )(xr, st, mu2, ls2, kernel_emb.astype(bf16), sign_emb, pt, pb, wcat,
      bcat, w1t, b1, w2t, b2)

    out = pl.pallas_call(
        functools.partial(_pass2_body, rows=r2, n_total=float(bsz * seq_len),
                          hidden=hidden),
        grid=(bsz // r2,),
        in_specs=[
            row3(r2, seq_len, hidden),
            pl.BlockSpec((bsz, 8, hidden), lambda b: (0, 0, 0)),
            row3(r2, 1, 2 * hidden),
            whole((1, hidden)), whole((1, hidden)), whole((1, hidden)),
        ],
        out_specs=row3(r2, 1, hidden),
        out_shape=jax.ShapeDtypeStruct((bsz, 1, hidden), f32),
        compiler_params=pltpu.CompilerParams(
            dimension_semantics=("arbitrary",),
            vmem_limit_bytes=56 * 1024 * 1024,
        ),
        name="traffic_node_pass2",
        interpret=interpret,
    )(h, stats, film, bng, bnb, aw)

    return out.reshape(bsz, hidden)


# speedup vs baseline: 1.1315x; 1.0142x over previous
import functools

import jax
import jax.numpy as jnp
from jax.experimental import pallas as pl
from jax.experimental.pallas import tpu as pltpu

_MIN_SIGMA = 1.0
_BN_EPS = 1e-5


def _pass1_body(xr_ref, st_ref, mu_ref, ls_ref, kemb_ref, semb_ref, pt_ref,
                pb_ref, wcat_ref, bcat_ref, w1t_ref, b1_ref, w2t_ref, b2_ref,
                h_ref, stats_ref, film_ref, seq_scr, im2_scr, *,
                rows, seq_len, emb_dim, hidden):
    f32 = jnp.float32
    bf16 = jnp.bfloat16
    sig = jax.nn.softplus(ls_ref[...]) + _MIN_SIGMA
    neg_half_inv = -0.5 / (sig * sig)
    stb = jnp.reshape(st_ref[...], (rows, st_ref.shape[2]))
    f1 = jnp.maximum(jnp.dot(stb.astype(bf16), w1t_ref[...],
                             preferred_element_type=f32)
                     + b1_ref[...], 0.0)
    film = jnp.dot(f1.astype(bf16), w2t_ref[...],
                   preferred_element_type=f32) + b2_ref[...]
    film_ref[...] = jnp.reshape(film, film_ref.shape)
    for r in range(rows):
        x = xr_ref[r]
        xa = jnp.abs(x)
        d = xa - mu_ref[...]
        w = jnp.exp(d * d * neg_half_inv)
        s = jnp.sum(w, axis=-1, keepdims=True)
        wn = w / (s + 1e-8)
        mag = jnp.dot(wn.astype(bf16), kemb_ref[...],
                      preferred_element_type=f32)
        s_emb = jnp.where(x < 0.0, semb_ref[1:2, :],
                          jnp.where(x > 0.0, semb_ref[2:3, :],
                                    semb_ref[0:1, :]))
        emb = mag + s_emb
        seq = jnp.dot(emb.astype(bf16), pt_ref[...],
                      preferred_element_type=f32) + pb_ref[...]
        seq_scr[r, 0:4, :] = jnp.zeros((4, emb_dim), f32)
        seq_scr[r, 4:4 + seq_len, :] = seq
        seq_scr[r, 4 + seq_len:8 + seq_len, :] = jnp.zeros((4, emb_dim), f32)
        for o in range(7):
            im2_scr[r, :, o * emb_dim:(o + 1) * emb_dim] = (
                seq_scr[r, 1 + o:1 + o + seq_len, :].astype(bf16))
        h = jnp.dot(im2_scr[r], wcat_ref[...], preferred_element_type=f32)
        h = jnp.maximum(h + bcat_ref[...], 0.0)
        h_ref[r] = h
        s1 = jnp.sum(h, axis=0, keepdims=True)
        s2 = jnp.sum(h * h, axis=0, keepdims=True)
        stats_ref[r] = jnp.concatenate(
            [s1, s2, jnp.zeros((6, hidden), f32)], axis=0)


def _pass2_body(h_ref, stats_ref, film_ref, bng_ref, bnb_ref, aw_ref,
                o_ref, *, rows, n_total, hidden):
    f32 = jnp.float32
    bf16 = jnp.bfloat16
    ps = jnp.sum(stats_ref[...], axis=0)
    s1 = ps[0:1, :]
    s2 = ps[1:2, :]
    inv_n = 1.0 / n_total
    mean = s1 * inv_n
    var = s2 * inv_n - mean * mean
    a = bng_ref[...] * jax.lax.rsqrt(var + _BN_EPS)
    bsh = bnb_ref[...] - mean * a
    for r in range(rows):
        f = film_ref[r]
        gamma = f[:, :hidden]
        beta = f[:, hidden:]
        a2 = a * (1.0 + gamma)
        b2 = bsh * (1.0 + gamma) + beta
        hm = h_ref[r] * a2 + b2
        hmb = hm.astype(bf16)
        scr = jax.lax.dot_general(aw_ref[...], hmb, (((1,), (1,)), ((), ())),
                                  preferred_element_type=f32)
        m = jnp.max(scr, axis=1, keepdims=True)
        e = jnp.exp(scr - m)
        den = jnp.sum(e, axis=1, keepdims=True)
        wgt = e / den
        wgt_c = jnp.transpose(wgt)
        pooled = jnp.sum(hm * wgt_c, axis=0, keepdims=True)
        o_ref[r] = pooled


@functools.partial(jax.jit, static_argnames=("interpret",))
def kernel(raw_seq, raw_stats, mu, log_sigma, kernel_emb, sign_emb, proj_w,
           proj_b, conv3_w, conv3_b, conv5_w, conv5_b, conv7_w, conv7_b,
           bn_gamma, bn_beta, stat_w1, stat_b1, stat_w2, stat_b2, attn_w,
           attn_b, interpret=False):
    f32 = jnp.float32
    bf16 = jnp.bfloat16
    bsz, seq_len = raw_seq.shape
    n_kern, emb_dim = kernel_emb.shape
    c3 = conv3_w.shape[0]
    c5 = conv5_w.shape[0]
    c7 = conv7_w.shape[0]
    hidden = c3 + c5 + c7
    stat_dim = raw_stats.shape[1]
    mlp_hid = stat_w1.shape[0]

    xr = raw_seq.reshape(bsz, seq_len, 1)
    st = raw_stats.reshape(bsz, 1, stat_dim)
    mu2 = mu.reshape(1, n_kern)
    ls2 = log_sigma.reshape(1, n_kern)
    pt = proj_w.T.astype(bf16)
    pb = proj_b.reshape(1, emb_dim)
    blocks = []
    for o in range(-3, 4):
        parts = []
        for w_c, k in ((conv3_w, 3), (conv5_w, 5), (conv7_w, 7)):
            p = k // 2
            t = o + p
            if 0 <= t < k:
                parts.append(w_c[:, :, t].T)
            else:
                parts.append(jnp.zeros((emb_dim, w_c.shape[0]), f32))
        blocks.append(jnp.concatenate(parts, axis=1))
    wcat = jnp.concatenate(blocks, axis=0).astype(bf16)
    bcat = jnp.concatenate([conv3_b, conv5_b, conv7_b]).reshape(1, hidden)
    w1t = stat_w1.T.astype(bf16)
    b1 = stat_b1.reshape(1, mlp_hid)
    w2t = stat_w2.T.astype(bf16)
    b2 = stat_b2.reshape(1, 2 * hidden)
    bng = bn_gamma.reshape(1, hidden)
    bnb = bn_beta.reshape(1, hidden)
    aw = attn_w.reshape(1, hidden).astype(bf16)

    r1 = 4
    r2 = 8
    whole = lambda shape: pl.BlockSpec(shape, lambda b: tuple(0 for _ in shape))
    row3 = lambda g, s1, s2: pl.BlockSpec((g, s1, s2), lambda b: (b, 0, 0))

    h, stats, film = pl.pallas_call(
        functools.partial(_pass1_body, rows=r1, seq_len=seq_len,
                          emb_dim=emb_dim, hidden=hidden),
        grid=(bsz // r1,),
        in_specs=[
            row3(r1, seq_len, 1),
            row3(r1, 1, stat_dim),
            whole((1, n_kern)), whole((1, n_kern)),
            whole((n_kern, emb_dim)), whole((3, emb_dim)),
            whole((emb_dim, emb_dim)), whole((1, emb_dim)),
            whole((7 * emb_dim, hidden)), whole((1, hidden)),
            whole((stat_dim, mlp_hid)), whole((1, mlp_hid)),
            whole((mlp_hid, 2 * hidden)), whole((1, 2 * hidden)),
        ],
        out_specs=[row3(r1, seq_len, hidden), row3(r1, 8, hidden),
                   row3(r1, 1, 2 * hidden)],
        out_shape=[
            jax.ShapeDtypeStruct((bsz, seq_len, hidden), f32),
            jax.ShapeDtypeStruct((bsz, 8, hidden), f32),
            jax.ShapeDtypeStruct((bsz, 1, 2 * hidden), f32),
        ],
        scratch_shapes=[
            pltpu.VMEM((r1, seq_len + 8, emb_dim), f32),
            pltpu.VMEM((r1, seq_len, 7 * emb_dim), bf16),
        ],
        compiler_params=pltpu.CompilerParams(
            dimension_semantics=("arbitrary",),
            vmem_limit_bytes=56 * 1024 * 1024,
        ),
        name="traffic_node_pass1",
        interpret=interpret,
    )(xr, st, mu2, ls2, kernel_emb.astype(bf16), sign_emb, pt, pb, wcat,
      bcat, w1t, b1, w2t, b2)

    out = pl.pallas_call(
        functools.partial(_pass2_body, rows=r2, n_total=float(bsz * seq_len),
                          hidden=hidden),
        grid=(bsz // r2,),
        in_specs=[
            row3(r2, seq_len, hidden),
            pl.BlockSpec((bsz, 8, hidden), lambda b: (0, 0, 0)),
            row3(r2, 1, 2 * hidden),
            whole((1, hidden)), whole((1, hidden)), whole((1, hidden)),
        ],
        out_specs=row3(r2, 1, hidden),
        out_shape=jax.ShapeDtypeStruct((bsz, 1, hidden), f32),
        compiler_params=pltpu.CompilerParams(
            dimension_semantics=("arbitrary",),
            vmem_limit_bytes=56 * 1024 * 1024,
        ),
        name="traffic_node_pass2",
        interpret=interpret,
    )(h, stats, film, bng, bnb, aw)

    return out.reshape(bsz, hidden)
